# Initial kernel scaffold; baseline (speedup 1.0000x reference)
import jax
import jax.numpy as jnp
from jax.experimental import pallas as pl


def kernel(x, edge_index, edge_index_out, W_emb, b_emb, Wc, bc, gamma, beta, W_fc, b_fc):
    raise NotImplementedError("write your pallas kernel here")


# speedup vs baseline: 6.0628x; 6.0628x over previous
import functools

import jax
import jax.numpy as jnp
from jax import lax
from jax.experimental import pallas as pl
from jax.experimental.pallas import tpu as pltpu
from jax.experimental.pallas import tpu_sc as plsc

N = 10000
E = 320000
D = 128
NLAYERS = 6
NPAD = 10240
NW = 32
K = 128
CH = 80
EPAD = NW * CH * K
KO = 128
CHO = EPAD // (NW * KO)
RPT = NPAD // 16

_mesh = plsc.VectorSubcoreMesh(core_axis_name="c", subcore_axis_name="s")


@functools.partial(
    pl.kernel,
    out_type=jax.ShapeDtypeStruct((2, 1, NPAD), jnp.float32),
    mesh=_mesh,
    scratch_types=[
        pltpu.VMEM((CHO, KO), jnp.int32),
        pltpu.VMEM((KO,), jnp.float32),
        pltpu.VMEM((RPT,), jnp.float32),
        pltpu.VMEM_SHARED((NPAD,), jnp.float32),
    ],
)
def _sc_degree(dst_hbm, out_hbm, dst_v, ones_v, buf_v, acc):
    c = lax.axis_index("c")
    s = lax.axis_index("s")
    wid = c * 16 + s
    pltpu.sync_copy(dst_hbm.at[wid], dst_v)
    for i in range(KO // 16):
        ones_v[pl.ds(i * 16, 16)] = jnp.ones((16,), jnp.float32)
    for i in range(RPT // 16):
        buf_v[pl.ds(i * 16, 16)] = jnp.zeros((16,), jnp.float32)
    roff = pl.multiple_of(s * RPT, 128)
    pltpu.sync_copy(buf_v, acc.at[pl.ds(roff, RPT)])
    plsc.subcore_barrier()

    def body(j, carry):
        pltpu.sync_copy(ones_v, acc.at[dst_v.at[j]], add=True)
        return carry

    lax.fori_loop(0, CHO, body, 0)
    plsc.subcore_barrier()
    pltpu.sync_copy(acc.at[pl.ds(roff, RPT)], buf_v)
    pltpu.sync_copy(buf_v, out_hbm.at[c, 0, pl.ds(roff, RPT)])


@functools.partial(
    pl.kernel,
    out_type=jax.ShapeDtypeStruct((2, NPAD, D), jnp.float32),
    mesh=_mesh,
    compiler_params=pltpu.CompilerParams(needs_layout_passes=False),
    scratch_types=[
        pltpu.VMEM((CH * K // 2,), jnp.int32),
        pltpu.VMEM((CH * K // 2,), jnp.int32),
        pltpu.VMEM((2, K), jnp.int32),
        pltpu.VMEM((2, K), jnp.int32),
        pltpu.VMEM((2, K, D), jnp.float32),
        pltpu.VMEM_SHARED((NPAD, D), jnp.float32),
        pltpu.SemaphoreType.DMA,
        pltpu.SemaphoreType.DMA,
    ],
)
def _sc_segsum(h_hbm, src_hbm, dst_hbm, out_hbm,
               src16_v, dst16_v, sidx, didx, rowbuf, acc, sem0, sem1):
    c = lax.axis_index("c")
    s = lax.axis_index("s")
    wid = c * 16 + s
    ioff = pl.multiple_of(wid * (CH * K // 2), 128)
    pltpu.sync_copy(src_hbm.at[pl.ds(ioff, CH * K // 2)], src16_v)
    pltpu.sync_copy(dst_hbm.at[pl.ds(ioff, CH * K // 2)], dst16_v)

    def cvt(j, b):
        for v32, ring in ((src16_v, sidx), (dst16_v, didx)):
            for g in range(K // 32):
                w = v32[pl.ds(j * (K // 2) + g * 16, 16)]
                ring[b, pl.ds(g * 32, 16)] = w & 0xFFFF
                ring[b, pl.ds(g * 32 + 16, 16)] = lax.shift_right_logical(
                    w, 16)

    def zb(t, carry):
        rowbuf[0, t // 8, pl.ds((t % 8) * 16, 16)] = jnp.zeros((16,),
                                                               jnp.float32)
        return carry

    lax.fori_loop(0, K * (D // 16), zb, 0)
    roff = pl.multiple_of(s * RPT, 128)
    for i in range(RPT // K):
        pltpu.sync_copy(rowbuf.at[0], acc.at[pl.ds(roff + i * K, K), :])
    plsc.subcore_barrier()

    sems = (sem0, sem1)
    cvt(0, 0)
    pltpu.async_copy(h_hbm.at[sidx.at[0]], rowbuf.at[0], sem0)
    cvt(1, 1)
    pltpu.async_copy(h_hbm.at[sidx.at[1]], rowbuf.at[1], sem1)

    def body(i, carry):
        j = i * 2
        for b in range(2):
            jj = j + b
            pltpu.make_async_copy(h_hbm.at[sidx.at[b]], rowbuf.at[b],
                                  sems[b]).wait()
            pltpu.sync_copy(rowbuf.at[b], acc.at[didx.at[b]], add=True)

            @pl.when(jj + 2 < CH)
            def _():
                cvt(jj + 2, b)
                pltpu.async_copy(h_hbm.at[sidx.at[b]], rowbuf.at[b],
                                 sems[b])

        return carry

    lax.fori_loop(0, CH // 2, body, 0)
    plsc.subcore_barrier()
    for i in range(RPT // K):
        pltpu.sync_copy(acc.at[pl.ds(roff + i * K, K), :], rowbuf.at[0])
        pltpu.sync_copy(rowbuf.at[0],
                        out_hbm.at[c, pl.ds(roff + i * K, K), :])


@functools.partial(
    pl.kernel,
    out_type=jax.ShapeDtypeStruct((EPAD * 2,), jnp.float32),
    mesh=_mesh,
    compiler_params=pltpu.CompilerParams(needs_layout_passes=False),
    scratch_types=[
        pltpu.VMEM((CHO, KO), jnp.int32),
        pltpu.VMEM((CHO, KO), jnp.int32),
        pltpu.VMEM((2 * NPAD,), jnp.float32),
        pltpu.VMEM((2 * NPAD,), jnp.float32),
        pltpu.VMEM((2 * KO,), jnp.float32),
    ],
)
def _sc_edgeout(p1_hbm, p2_hbm, s_hbm, d_hbm, out_hbm,
                s_v, d_v, p1_v, p2_v, obuf):
    c = lax.axis_index("c")
    s = lax.axis_index("s")
    wid = c * 16 + s
    pltpu.sync_copy(s_hbm.at[wid], s_v)
    pltpu.sync_copy(d_hbm.at[wid], d_v)
    pltpu.sync_copy(p1_hbm, p1_v)
    pltpu.sync_copy(p2_hbm, p2_v)
    iota16 = lax.iota(jnp.int32, 16)

    def body(j, carry):
        for g in range(KO // 16):
            sseg = s_v[j, pl.ds(g * 16, 16)]
            dseg = d_v[j, pl.ds(g * 16, 16)]
            a0 = plsc.load_gather(p1_v, [sseg * 2])
            a1 = plsc.load_gather(p1_v, [sseg * 2 + 1])
            b0 = plsc.load_gather(p2_v, [dseg * 2])
            b1 = plsc.load_gather(p2_v, [dseg * 2 + 1])
            pos = iota16 * 2 + g * 32
            plsc.store_scatter(obuf, [pos], a0 + b0)
            plsc.store_scatter(obuf, [pos + 1], a1 + b1)
        base = pl.multiple_of((wid * CHO + j) * (2 * KO), 256)
        pltpu.sync_copy(obuf, out_hbm.at[pl.ds(base, 2 * KO)])
        return carry

    lax.fori_loop(0, CHO, body, 0)


def _tc_embed_body(x_ref, wemb_ref, bemb_ref, wc0_ref, degt_ref,
                   xe_ref, dinvb_ref, h_ref):
    i = pl.program_id(0)
    xe = jnp.dot(x_ref[...], wemb_ref[...],
                 preferred_element_type=jnp.float32) + bemb_ref[...]
    rows = i * 256 + lax.broadcasted_iota(jnp.int32, (256, 1), 0)
    valid = rows < N
    xe = jnp.where(valid, xe, 0.0)
    deg = degt_ref[...]
    dtot = deg[:, 0:1] + deg[:, 1:2] + 1.0
    dinv = jnp.where(valid, 1.0 / jnp.sqrt(dtot), 0.0)
    dinvb = jnp.broadcast_to(dinv, (256, D))
    xe_ref[...] = xe
    dinvb_ref[...] = dinvb
    h_ref[...] = jnp.dot(xe * dinvb, wc0_ref[...],
                         preferred_element_type=jnp.float32)


_tc_embed = pl.pallas_call(
    _tc_embed_body,
    grid=(NPAD // 256,),
    in_specs=[
        pl.BlockSpec((256, D), lambda i: (i, 0)),
        pl.BlockSpec((D, D), lambda i: (0, 0)),
        pl.BlockSpec((1, D), lambda i: (0, 0)),
        pl.BlockSpec((D, D), lambda i: (0, 0)),
        pl.BlockSpec((256, 2), lambda i: (i, 0)),
    ],
    out_specs=[
        pl.BlockSpec((256, D), lambda i: (i, 0)),
        pl.BlockSpec((256, D), lambda i: (i, 0)),
        pl.BlockSpec((256, D), lambda i: (i, 0)),
    ],
    out_shape=[
        jax.ShapeDtypeStruct((NPAD, D), jnp.float32),
        jax.ShapeDtypeStruct((NPAD, D), jnp.float32),
        jax.ShapeDtypeStruct((NPAD, D), jnp.float32),
    ],
)


def _combine_core(s0_ref, s1_ref, h_ref, xe_ref, dinvb_ref,
                  bc_ref, g_ref, b_ref):
    agg = (s0_ref[...] + s1_ref[...] + h_ref[...]) * dinvb_ref[...] \
        + bc_ref[...]
    rows = lax.broadcasted_iota(jnp.int32, (NPAD, 1), 0)
    valid = rows < N
    agg = jnp.where(valid, agg, 0.0)
    mu = jnp.sum(agg, axis=0, keepdims=True) / N
    var = jnp.sum(agg * agg, axis=0, keepdims=True) / N - mu * mu
    hbn = (agg - mu) * (1.0 / jnp.sqrt(var + 1e-5)) * g_ref[...] + b_ref[...]
    xen = xe_ref[...] + jnp.maximum(hbn, 0.0)
    return jnp.where(valid, xen, 0.0)


def _tc_combine_body(s0_ref, s1_ref, h_ref, xe_ref, dinvb_ref,
                     bc_ref, g_ref, b_ref, wn_ref, xe_out, hn_out):
    xen = _combine_core(s0_ref, s1_ref, h_ref, xe_ref, dinvb_ref,
                        bc_ref, g_ref, b_ref)
    xe_out[...] = xen
    hn_out[...] = jnp.dot(xen * dinvb_ref[...], wn_ref[...],
                          preferred_element_type=jnp.float32)


_tc_combine = pl.pallas_call(
    _tc_combine_body,
    out_shape=[
        jax.ShapeDtypeStruct((NPAD, D), jnp.float32),
        jax.ShapeDtypeStruct((NPAD, D), jnp.float32),
    ],
)


def _tc_combine_last_body(s0_ref, s1_ref, h_ref, xe_ref, dinvb_ref,
                          bc_ref, g_ref, b_ref, wfc_ref, p4_out):
    xen = _combine_core(s0_ref, s1_ref, h_ref, xe_ref, dinvb_ref,
                        bc_ref, g_ref, b_ref)
    p4_out[...] = jnp.dot(xen, wfc_ref[...],
                          preferred_element_type=jnp.float32)


_tc_combine_last = pl.pallas_call(
    _tc_combine_last_body,
    out_shape=jax.ShapeDtypeStruct((NPAD, 4), jnp.float32),
)


def _pack16(a):
    return a[0::2] | (a[1::2] << 16)


def _pad_idx(a, kk):
    a = jnp.concatenate([a, jnp.full((EPAD - E,), NPAD - 1, jnp.int32)])
    return a.reshape(NW, EPAD // (NW * kk), kk)


def kernel(x, edge_index, edge_index_out, W_emb, b_emb, Wc, bc, gamma, beta,
           W_fc, b_fc):
    x_pad = jnp.pad(x, ((0, NPAD - N), (0, 0)))
    src3 = _pack16(_pad_idx(edge_index[0], K).reshape(-1))
    dst3 = _pack16(_pad_idx(edge_index[1], K).reshape(-1))
    dst3w = _pad_idx(edge_index[1], KO)
    so3 = _pad_idx(edge_index_out[0], KO)
    do3 = _pad_idx(edge_index_out[1], KO)

    deg2 = _sc_degree(dst3w)
    degt = jnp.transpose(deg2[:, 0, :])

    xe, dinvb, h = _tc_embed(x_pad, W_emb, b_emb.reshape(1, D), Wc[0], degt)

    for i in range(NLAYERS):
        s2 = _sc_segsum(h, src3, dst3)
        bci = bc[i].reshape(1, D)
        gi = gamma[i].reshape(1, D)
        bi = beta[i].reshape(1, D)
        if i < NLAYERS - 1:
            xe, h = _tc_combine(s2[0], s2[1], h, xe, dinvb, bci, gi, bi,
                                Wc[i + 1])
        else:
            wfc4 = jnp.concatenate([W_fc[:D], W_fc[D:]], axis=1)
            p4 = _tc_combine_last(s2[0], s2[1], h, xe, dinvb, bci, gi, bi,
                                  wfc4)

    p1f = (p4[:, 0:2] + b_fc).reshape(-1)
    p2f = p4[:, 2:4].reshape(-1)
    outf = _sc_edgeout(p1f, p2f, so3, do3)
    return outf[: E * 2].reshape(E, 2)


# speedup vs baseline: 15.2685x; 2.5184x over previous
import functools

import jax
import jax.numpy as jnp
from jax import lax
from jax.experimental import pallas as pl
from jax.experimental.pallas import tpu as pltpu
from jax.experimental.pallas import tpu_sc as plsc

N = 10000
E = 320000
D = 128
NLAYERS = 6
NPAD = 10240
NW = 32
K = 128
CH = 80
EPAD = NW * CH * K
KO = 128
CHO = EPAD // (NW * KO)
RPT = NPAD // 16

_mesh = plsc.VectorSubcoreMesh(core_axis_name="c", subcore_axis_name="s")


@functools.partial(
    pl.kernel,
    out_type=jax.ShapeDtypeStruct((2, 1, NPAD), jnp.float32),
    mesh=_mesh,
    scratch_types=[
        pltpu.VMEM((CHO, KO), jnp.int32),
        pltpu.VMEM((KO,), jnp.float32),
        pltpu.VMEM((RPT,), jnp.float32),
        pltpu.VMEM_SHARED((NPAD,), jnp.float32),
    ],
)
def _sc_degree(dst_hbm, out_hbm, dst_v, ones_v, buf_v, acc):
    c = lax.axis_index("c")
    s = lax.axis_index("s")
    wid = c * 16 + s
    pltpu.sync_copy(dst_hbm.at[wid], dst_v)
    for i in range(KO // 16):
        ones_v[pl.ds(i * 16, 16)] = jnp.ones((16,), jnp.float32)
    for i in range(RPT // 16):
        buf_v[pl.ds(i * 16, 16)] = jnp.zeros((16,), jnp.float32)
    roff = pl.multiple_of(s * RPT, 128)
    pltpu.sync_copy(buf_v, acc.at[pl.ds(roff, RPT)])
    plsc.subcore_barrier()

    def body(j, carry):
        pltpu.sync_copy(ones_v, acc.at[dst_v.at[j]], add=True)
        return carry

    lax.fori_loop(0, CHO, body, 0)
    plsc.subcore_barrier()
    pltpu.sync_copy(acc.at[pl.ds(roff, RPT)], buf_v)
    pltpu.sync_copy(buf_v, out_hbm.at[c, 0, pl.ds(roff, RPT)])


@functools.partial(
    pl.kernel,
    out_type=jax.ShapeDtypeStruct((2, NPAD, D), jnp.float32),
    mesh=_mesh,
    compiler_params=pltpu.CompilerParams(needs_layout_passes=False),
    scratch_types=[
        pltpu.VMEM((CH * K // 2,), jnp.int32),
        pltpu.VMEM((CH * K // 2,), jnp.int32),
        pltpu.VMEM((2, K), jnp.int32),
        pltpu.VMEM((2, K), jnp.int32),
        pltpu.VMEM((2, K, D), jnp.float32),
        pltpu.VMEM_SHARED((NPAD, D), jnp.float32),
        pltpu.SemaphoreType.DMA,
        pltpu.SemaphoreType.DMA,
    ],
)
def _sc_segsum(h_hbm, src_hbm, dst_hbm, out_hbm,
               src16_v, dst16_v, sidx, didx, rowbuf, acc, sem0, sem1):
    c = lax.axis_index("c")
    s = lax.axis_index("s")
    wid = c * 16 + s
    ioff = pl.multiple_of(wid * (CH * K // 2), 128)
    pltpu.sync_copy(src_hbm.at[pl.ds(ioff, CH * K // 2)], src16_v)
    pltpu.sync_copy(dst_hbm.at[pl.ds(ioff, CH * K // 2)], dst16_v)

    def cvt(j, b):
        for v32, ring in ((src16_v, sidx), (dst16_v, didx)):
            for g in range(K // 32):
                w = v32[pl.ds(j * (K // 2) + g * 16, 16)]
                ring[b, pl.ds(g * 32, 16)] = w & 0xFFFF
                ring[b, pl.ds(g * 32 + 16, 16)] = lax.shift_right_logical(
                    w, 16)

    def zb(t, carry):
        rowbuf[0, t // 8, pl.ds((t % 8) * 16, 16)] = jnp.zeros((16,),
                                                               jnp.float32)
        return carry

    lax.fori_loop(0, K * (D // 16), zb, 0)
    roff = pl.multiple_of(s * RPT, 128)
    for i in range(RPT // K):
        pltpu.sync_copy(rowbuf.at[0], acc.at[pl.ds(roff + i * K, K), :])
    plsc.subcore_barrier()

    sems = (sem0, sem1)
    cvt(0, 0)
    pltpu.async_copy(h_hbm.at[sidx.at[0]], rowbuf.at[0], sem0)
    cvt(1, 1)
    pltpu.async_copy(h_hbm.at[sidx.at[1]], rowbuf.at[1], sem1)

    def body(i, carry):
        j = i * 2
        for b in range(2):
            jj = j + b
            pltpu.make_async_copy(h_hbm.at[sidx.at[b]], rowbuf.at[b],
                                  sems[b]).wait()
            pltpu.sync_copy(rowbuf.at[b], acc.at[didx.at[b]], add=True)

            @pl.when(jj + 2 < CH)
            def _():
                cvt(jj + 2, b)
                pltpu.async_copy(h_hbm.at[sidx.at[b]], rowbuf.at[b],
                                 sems[b])

        return carry

    lax.fori_loop(0, CH // 2, body, 0)
    plsc.subcore_barrier()
    for i in range(RPT // K):
        pltpu.sync_copy(acc.at[pl.ds(roff + i * K, K), :], rowbuf.at[0])
        pltpu.sync_copy(rowbuf.at[0],
                        out_hbm.at[c, pl.ds(roff + i * K, K), :])


@functools.partial(
    pl.kernel,
    out_type=jax.ShapeDtypeStruct((EPAD * 2,), jnp.float32),
    mesh=_mesh,
    compiler_params=pltpu.CompilerParams(needs_layout_passes=False),
    scratch_types=[
        pltpu.VMEM((CHO, KO), jnp.int32),
        pltpu.VMEM((CHO, KO), jnp.int32),
        pltpu.VMEM((2 * NPAD,), jnp.float32),
        pltpu.VMEM((2 * NPAD,), jnp.float32),
        pltpu.VMEM((2 * KO,), jnp.float32),
    ],
)
def _sc_edgeout(p1_hbm, p2_hbm, s_hbm, d_hbm, out_hbm,
                s_v, d_v, p1_v, p2_v, obuf):
    c = lax.axis_index("c")
    s = lax.axis_index("s")
    wid = c * 16 + s
    pltpu.sync_copy(s_hbm.at[wid], s_v)
    pltpu.sync_copy(d_hbm.at[wid], d_v)
    pltpu.sync_copy(p1_hbm, p1_v)
    pltpu.sync_copy(p2_hbm, p2_v)
    iota16 = lax.iota(jnp.int32, 16)

    def body(j, carry):
        for g in range(KO // 16):
            sseg = s_v[j, pl.ds(g * 16, 16)]
            dseg = d_v[j, pl.ds(g * 16, 16)]
            a0 = plsc.load_gather(p1_v, [sseg * 2])
            a1 = plsc.load_gather(p1_v, [sseg * 2 + 1])
            b0 = plsc.load_gather(p2_v, [dseg * 2])
            b1 = plsc.load_gather(p2_v, [dseg * 2 + 1])
            pos = iota16 * 2 + g * 32
            plsc.store_scatter(obuf, [pos], a0 + b0)
            plsc.store_scatter(obuf, [pos + 1], a1 + b1)
        base = pl.multiple_of((wid * CHO + j) * (2 * KO), 256)
        pltpu.sync_copy(obuf, out_hbm.at[pl.ds(base, 2 * KO)])
        return carry

    lax.fori_loop(0, CHO, body, 0)


def _tc_embed_body(x_ref, wemb_ref, bemb_ref, wc0_ref, degt_ref,
                   xe_ref, dinvb_ref, h_ref):
    i = pl.program_id(0)
    xe = jnp.dot(x_ref[...], wemb_ref[...],
                 preferred_element_type=jnp.float32) + bemb_ref[...]
    rows = i * 256 + lax.broadcasted_iota(jnp.int32, (256, 1), 0)
    valid = rows < N
    xe = jnp.where(valid, xe, 0.0)
    deg = degt_ref[...]
    dtot = deg[:, 0:1] + deg[:, 1:2] + 1.0
    dinv = jnp.where(valid, 1.0 / jnp.sqrt(dtot), 0.0)
    dinvb = jnp.broadcast_to(dinv, (256, D))
    xe_ref[...] = xe
    dinvb_ref[...] = dinvb
    h_ref[...] = jnp.dot(xe * dinvb, wc0_ref[...],
                         preferred_element_type=jnp.float32)


_tc_embed = pl.pallas_call(
    _tc_embed_body,
    grid=(NPAD // 256,),
    in_specs=[
        pl.BlockSpec((256, D), lambda i: (i, 0)),
        pl.BlockSpec((D, D), lambda i: (0, 0)),
        pl.BlockSpec((1, D), lambda i: (0, 0)),
        pl.BlockSpec((D, D), lambda i: (0, 0)),
        pl.BlockSpec((256, 2), lambda i: (i, 0)),
    ],
    out_specs=[
        pl.BlockSpec((256, D), lambda i: (i, 0)),
        pl.BlockSpec((256, D), lambda i: (i, 0)),
        pl.BlockSpec((256, D), lambda i: (i, 0)),
    ],
    out_shape=[
        jax.ShapeDtypeStruct((NPAD, D), jnp.float32),
        jax.ShapeDtypeStruct((NPAD, D), jnp.float32),
        jax.ShapeDtypeStruct((NPAD, D), jnp.float32),
    ],
)


def _combine_core(s0_ref, s1_ref, h_ref, xe_ref, dinvb_ref,
                  bc_ref, g_ref, b_ref):
    agg = (s0_ref[...] + s1_ref[...] + h_ref[...]) * dinvb_ref[...] \
        + bc_ref[...]
    rows = lax.broadcasted_iota(jnp.int32, (NPAD, 1), 0)
    valid = rows < N
    agg = jnp.where(valid, agg, 0.0)
    mu = jnp.sum(agg, axis=0, keepdims=True) / N
    var = jnp.sum(agg * agg, axis=0, keepdims=True) / N - mu * mu
    hbn = (agg - mu) * (1.0 / jnp.sqrt(var + 1e-5)) * g_ref[...] + b_ref[...]
    xen = xe_ref[...] + jnp.maximum(hbn, 0.0)
    return jnp.where(valid, xen, 0.0)


def _tc_combine_body(s0_ref, s1_ref, h_ref, xe_ref, dinvb_ref,
                     bc_ref, g_ref, b_ref, wn_ref, xe_out, hn_out):
    xen = _combine_core(s0_ref, s1_ref, h_ref, xe_ref, dinvb_ref,
                        bc_ref, g_ref, b_ref)
    xe_out[...] = xen
    hn_out[...] = jnp.dot(xen * dinvb_ref[...], wn_ref[...],
                          preferred_element_type=jnp.float32)


_tc_combine = pl.pallas_call(
    _tc_combine_body,
    out_shape=[
        jax.ShapeDtypeStruct((NPAD, D), jnp.float32),
        jax.ShapeDtypeStruct((NPAD, D), jnp.float32),
    ],
)


def _tc_combine_last_body(s0_ref, s1_ref, h_ref, xe_ref, dinvb_ref,
                          bc_ref, g_ref, b_ref, wfc_ref, p4_out):
    xen = _combine_core(s0_ref, s1_ref, h_ref, xe_ref, dinvb_ref,
                        bc_ref, g_ref, b_ref)
    p4_out[...] = jnp.dot(xen, wfc_ref[...],
                          preferred_element_type=jnp.float32)


_tc_combine_last = pl.pallas_call(
    _tc_combine_last_body,
    out_shape=jax.ShapeDtypeStruct((NPAD, 4), jnp.float32),
)


def _pack16(a):
    return a[0::2] | (a[1::2] << 16)


def _pad_idx(a, kk):
    fill = N + (jnp.arange(EPAD - E, dtype=jnp.int32) % (NPAD - N))
    a = jnp.concatenate([a, fill])
    return a.reshape(NW, EPAD // (NW * kk), kk)


def kernel(x, edge_index, edge_index_out, W_emb, b_emb, Wc, bc, gamma, beta,
           W_fc, b_fc):
    x_pad = jnp.pad(x, ((0, NPAD - N), (0, 0)))
    src3 = _pack16(_pad_idx(edge_index[0], K).reshape(-1))
    dst3 = _pack16(_pad_idx(edge_index[1], K).reshape(-1))
    dst3w = _pad_idx(edge_index[1], KO)
    so3 = _pad_idx(edge_index_out[0], KO)
    do3 = _pad_idx(edge_index_out[1], KO)

    deg2 = _sc_degree(dst3w)
    degt = jnp.transpose(deg2[:, 0, :])

    xe, dinvb, h = _tc_embed(x_pad, W_emb, b_emb.reshape(1, D), Wc[0], degt)

    for i in range(NLAYERS):
        s2 = _sc_segsum(h, src3, dst3)
        bci = bc[i].reshape(1, D)
        gi = gamma[i].reshape(1, D)
        bi = beta[i].reshape(1, D)
        if i < NLAYERS - 1:
            xe, h = _tc_combine(s2[0], s2[1], h, xe, dinvb, bci, gi, bi,
                                Wc[i + 1])
        else:
            wfc4 = jnp.concatenate([W_fc[:D], W_fc[D:]], axis=1)
            p4 = _tc_combine_last(s2[0], s2[1], h, xe, dinvb, bci, gi, bi,
                                  wfc4)

    p1f = (p4[:, 0:2] + b_fc).reshape(-1)
    p2f = p4[:, 2:4].reshape(-1)
    outf = _sc_edgeout(p1f, p2f, so3, do3)
    return outf[: E * 2].reshape(E, 2)


# speedup vs baseline: 18.0977x; 1.1853x over previous
import functools

import jax
import jax.numpy as jnp
from jax import lax
from jax.experimental import pallas as pl
from jax.experimental.pallas import tpu as pltpu
from jax.experimental.pallas import tpu_sc as plsc

N = 10000
E = 320000
D = 128
NLAYERS = 6
NPAD = 10240
NW = 32
K = 128
CH = 80
EPAD = NW * CH * K
KO = 128
CHO = EPAD // (NW * KO)
RPT = NPAD // 16

_mesh = plsc.VectorSubcoreMesh(core_axis_name="c", subcore_axis_name="s")


@functools.partial(
    pl.kernel,
    out_type=jax.ShapeDtypeStruct((2, 1, NPAD), jnp.float32),
    mesh=_mesh,
    scratch_types=[
        pltpu.VMEM((CHO, KO), jnp.int32),
        pltpu.VMEM((KO,), jnp.float32),
        pltpu.VMEM((RPT,), jnp.float32),
        pltpu.VMEM_SHARED((NPAD,), jnp.float32),
    ],
)
def _sc_degree(dst_hbm, out_hbm, dst_v, ones_v, buf_v, acc):
    c = lax.axis_index("c")
    s = lax.axis_index("s")
    wid = c * 16 + s
    pltpu.sync_copy(dst_hbm.at[wid], dst_v)
    for i in range(KO // 16):
        ones_v[pl.ds(i * 16, 16)] = jnp.ones((16,), jnp.float32)
    for i in range(RPT // 16):
        buf_v[pl.ds(i * 16, 16)] = jnp.zeros((16,), jnp.float32)
    roff = pl.multiple_of(s * RPT, 128)
    pltpu.sync_copy(buf_v, acc.at[pl.ds(roff, RPT)])
    plsc.subcore_barrier()

    def body(j, carry):
        pltpu.sync_copy(ones_v, acc.at[dst_v.at[j]], add=True)
        return carry

    lax.fori_loop(0, CHO, body, 0)
    plsc.subcore_barrier()
    pltpu.sync_copy(acc.at[pl.ds(roff, RPT)], buf_v)
    pltpu.sync_copy(buf_v, out_hbm.at[c, 0, pl.ds(roff, RPT)])


@functools.partial(
    pl.kernel,
    out_type=(jax.ShapeDtypeStruct((NPAD, D), jnp.float32),
              jax.ShapeDtypeStruct((NPAD, D), jnp.float32)),
    mesh=_mesh,
    compiler_params=pltpu.CompilerParams(needs_layout_passes=False),
    scratch_types=[
        pltpu.VMEM((CH * K // 2,), jnp.int32),
        pltpu.VMEM((CH * K // 2,), jnp.int32),
        pltpu.VMEM((2, K), jnp.int32),
        pltpu.VMEM((2, K), jnp.int32),
        pltpu.VMEM((2, K, D), jnp.float32),
        pltpu.VMEM_SHARED((NPAD, D), jnp.float32),
        pltpu.SemaphoreType.DMA,
        pltpu.SemaphoreType.DMA,
    ],
)
def _sc_segsum(h_hbm, src_hbm, dst_hbm, out0_hbm, out1_hbm,
               src16_v, dst16_v, sidx, didx, rowbuf, acc, sem0, sem1):
    c = lax.axis_index("c")
    s = lax.axis_index("s")
    wid = c * 16 + s
    ioff = pl.multiple_of(wid * (CH * K // 2), 128)
    pltpu.sync_copy(src_hbm.at[pl.ds(ioff, CH * K // 2)], src16_v)
    pltpu.sync_copy(dst_hbm.at[pl.ds(ioff, CH * K // 2)], dst16_v)

    def cvt(j, b):
        for v32, ring in ((src16_v, sidx), (dst16_v, didx)):
            for g in range(K // 32):
                w = v32[pl.ds(j * (K // 2) + g * 16, 16)]
                ring[b, pl.ds(g * 32, 16)] = w & 0xFFFF
                ring[b, pl.ds(g * 32 + 16, 16)] = lax.shift_right_logical(
                    w, 16)

    def zb(t, carry):
        rowbuf[0, t // 8, pl.ds((t % 8) * 16, 16)] = jnp.zeros((16,),
                                                               jnp.float32)
        return carry

    lax.fori_loop(0, K * (D // 16), zb, 0)
    roff = pl.multiple_of(s * RPT, 128)
    for i in range(RPT // K):
        pltpu.sync_copy(rowbuf.at[0], acc.at[pl.ds(roff + i * K, K), :])
    plsc.subcore_barrier()

    sems = (sem0, sem1)
    cvt(0, 0)
    pltpu.async_copy(h_hbm.at[sidx.at[0]], rowbuf.at[0], sem0)
    cvt(1, 1)
    pltpu.async_copy(h_hbm.at[sidx.at[1]], rowbuf.at[1], sem1)

    def body(i, carry):
        j = i * 2
        for b in range(2):
            jj = j + b
            pltpu.make_async_copy(h_hbm.at[sidx.at[b]], rowbuf.at[b],
                                  sems[b]).wait()
            pltpu.sync_copy(rowbuf.at[b], acc.at[didx.at[b]], add=True)

            @pl.when(jj + 2 < CH)
            def _():
                cvt(jj + 2, b)
                pltpu.async_copy(h_hbm.at[sidx.at[b]], rowbuf.at[b],
                                 sems[b])

        return carry

    lax.fori_loop(0, CH // 2, body, 0)
    plsc.subcore_barrier()
    for i in range(RPT // K):
        pltpu.sync_copy(acc.at[pl.ds(roff + i * K, K), :], rowbuf.at[0])

        @pl.when(c == 0)
        def _():
            pltpu.sync_copy(rowbuf.at[0],
                            out0_hbm.at[pl.ds(roff + i * K, K), :])

        @pl.when(c == 1)
        def _():
            pltpu.sync_copy(rowbuf.at[0],
                            out1_hbm.at[pl.ds(roff + i * K, K), :])


@functools.partial(
    pl.kernel,
    out_type=(jax.ShapeDtypeStruct((EPAD,), jnp.float32),
              jax.ShapeDtypeStruct((EPAD,), jnp.float32)),
    mesh=_mesh,
    compiler_params=pltpu.CompilerParams(needs_layout_passes=False),
    scratch_types=[
        pltpu.VMEM((CHO, KO), jnp.int32),
        pltpu.VMEM((CHO, KO), jnp.int32),
        pltpu.VMEM((2 * NPAD,), jnp.float32),
        pltpu.VMEM((2 * NPAD,), jnp.float32),
        pltpu.VMEM((KO,), jnp.float32),
        pltpu.VMEM((KO,), jnp.float32),
    ],
)
def _sc_edgeout(p1_hbm, p2_hbm, s_hbm, d_hbm, out0_hbm, out1_hbm,
                s_v, d_v, p1_v, p2_v, obuf0, obuf1):
    c = lax.axis_index("c")
    s = lax.axis_index("s")
    wid = c * 16 + s
    pltpu.sync_copy(s_hbm.at[wid], s_v)
    pltpu.sync_copy(d_hbm.at[wid], d_v)
    pltpu.sync_copy(p1_hbm, p1_v)
    pltpu.sync_copy(p2_hbm, p2_v)

    def body(j, carry):
        for g in range(KO // 16):
            sseg = s_v[j, pl.ds(g * 16, 16)]
            dseg = d_v[j, pl.ds(g * 16, 16)]
            a0 = plsc.load_gather(p1_v, [sseg * 2])
            a1 = plsc.load_gather(p1_v, [sseg * 2 + 1])
            b0 = plsc.load_gather(p2_v, [dseg * 2])
            b1 = plsc.load_gather(p2_v, [dseg * 2 + 1])
            obuf0[pl.ds(g * 16, 16)] = a0 + b0
            obuf1[pl.ds(g * 16, 16)] = a1 + b1
        base = pl.multiple_of((wid * CHO + j) * KO, 128)
        pltpu.sync_copy(obuf0, out0_hbm.at[pl.ds(base, KO)])
        pltpu.sync_copy(obuf1, out1_hbm.at[pl.ds(base, KO)])
        return carry

    lax.fori_loop(0, CHO, body, 0)


def _tc_embed_body(x_ref, wemb_ref, bemb_ref, wc0_ref, degt_ref,
                   xe_ref, dinvb_ref, h_ref):
    i = pl.program_id(0)
    xe = jnp.dot(x_ref[...], wemb_ref[...],
                 preferred_element_type=jnp.float32) + bemb_ref[...]
    rows = i * 256 + lax.broadcasted_iota(jnp.int32, (256, 1), 0)
    valid = rows < N
    xe = jnp.where(valid, xe, 0.0)
    deg = degt_ref[...]
    dtot = deg[:, 0:1] + deg[:, 1:2] + 1.0
    dinv = jnp.where(valid, 1.0 / jnp.sqrt(dtot), 0.0)
    dinvb = jnp.broadcast_to(dinv, (256, D))
    xe_ref[...] = xe
    dinvb_ref[...] = dinvb
    h_ref[...] = jnp.dot(xe * dinvb, wc0_ref[...],
                         preferred_element_type=jnp.float32)


_tc_embed = pl.pallas_call(
    _tc_embed_body,
    grid=(NPAD // 256,),
    in_specs=[
        pl.BlockSpec((256, D), lambda i: (i, 0)),
        pl.BlockSpec((D, D), lambda i: (0, 0)),
        pl.BlockSpec((1, D), lambda i: (0, 0)),
        pl.BlockSpec((D, D), lambda i: (0, 0)),
        pl.BlockSpec((256, 2), lambda i: (i, 0)),
    ],
    out_specs=[
        pl.BlockSpec((256, D), lambda i: (i, 0)),
        pl.BlockSpec((256, D), lambda i: (i, 0)),
        pl.BlockSpec((256, D), lambda i: (i, 0)),
    ],
    out_shape=[
        jax.ShapeDtypeStruct((NPAD, D), jnp.float32),
        jax.ShapeDtypeStruct((NPAD, D), jnp.float32),
        jax.ShapeDtypeStruct((NPAD, D), jnp.float32),
    ],
)


def _combine_core(s0_ref, s1_ref, h_ref, xe_ref, dinvb_ref,
                  bc_ref, g_ref, b_ref):
    agg = (s0_ref[...] + s1_ref[...] + h_ref[...]) * dinvb_ref[...] \
        + bc_ref[...]
    rows = lax.broadcasted_iota(jnp.int32, (NPAD, 1), 0)
    valid = rows < N
    agg = jnp.where(valid, agg, 0.0)
    mu = jnp.sum(agg, axis=0, keepdims=True) / N
    var = jnp.sum(agg * agg, axis=0, keepdims=True) / N - mu * mu
    hbn = (agg - mu) * (1.0 / jnp.sqrt(var + 1e-5)) * g_ref[...] + b_ref[...]
    xen = xe_ref[...] + jnp.maximum(hbn, 0.0)
    return jnp.where(valid, xen, 0.0)


def _tc_combine_body(s0_ref, s1_ref, h_ref, xe_ref, dinvb_ref,
                     bc_ref, g_ref, b_ref, wn_ref, xe_out, hn_out):
    xen = _combine_core(s0_ref, s1_ref, h_ref, xe_ref, dinvb_ref,
                        bc_ref, g_ref, b_ref)
    xe_out[...] = xen
    hn_out[...] = jnp.dot(xen * dinvb_ref[...], wn_ref[...],
                          preferred_element_type=jnp.float32)


_tc_combine = pl.pallas_call(
    _tc_combine_body,
    out_shape=[
        jax.ShapeDtypeStruct((NPAD, D), jnp.float32),
        jax.ShapeDtypeStruct((NPAD, D), jnp.float32),
    ],
)


def _tc_combine_last_body(s0_ref, s1_ref, h_ref, xe_ref, dinvb_ref,
                          bc_ref, g_ref, b_ref, wfc_ref, p4_out):
    xen = _combine_core(s0_ref, s1_ref, h_ref, xe_ref, dinvb_ref,
                        bc_ref, g_ref, b_ref)
    p4_out[...] = jnp.dot(xen, wfc_ref[...],
                          preferred_element_type=jnp.float32)


_tc_combine_last = pl.pallas_call(
    _tc_combine_last_body,
    out_shape=jax.ShapeDtypeStruct((NPAD, 4), jnp.float32),
)


def _pack16(a):
    return lax.bitcast_convert_type(
        a.astype(jnp.int16).reshape(-1, 2), jnp.int32)


def _pad_idx(a, kk):
    fill = N + (jnp.arange(EPAD - E, dtype=jnp.int32) % (NPAD - N))
    a = jnp.concatenate([a, fill])
    return a.reshape(NW, EPAD // (NW * kk), kk)


def kernel(x, edge_index, edge_index_out, W_emb, b_emb, Wc, bc, gamma, beta,
           W_fc, b_fc):
    x_pad = jnp.pad(x, ((0, NPAD - N), (0, 0)))
    src3 = _pack16(_pad_idx(edge_index[0], K).reshape(-1))
    dst3 = _pack16(_pad_idx(edge_index[1], K).reshape(-1))
    dst3w = _pad_idx(edge_index[1], KO)
    so3 = _pad_idx(edge_index_out[0], KO)
    do3 = _pad_idx(edge_index_out[1], KO)

    deg2 = _sc_degree(dst3w)
    degt = jnp.transpose(deg2[:, 0, :])

    xe, dinvb, h = _tc_embed(x_pad, W_emb, b_emb.reshape(1, D), Wc[0], degt)

    for i in range(NLAYERS):
        s0, s1 = _sc_segsum(h, src3, dst3)
        bci = bc[i].reshape(1, D)
        gi = gamma[i].reshape(1, D)
        bi = beta[i].reshape(1, D)
        if i < NLAYERS - 1:
            xe, h = _tc_combine(s0, s1, h, xe, dinvb, bci, gi, bi,
                                Wc[i + 1])
        else:
            wfc4 = jnp.concatenate([W_fc[:D], W_fc[D:]], axis=1)
            p4 = _tc_combine_last(s0, s1, h, xe, dinvb, bci, gi, bi,
                                  wfc4)

    p1f = (p4[:, 0:2] + b_fc).reshape(-1)
    p2f = p4[:, 2:4].reshape(-1)
    o0, o1 = _sc_edgeout(p1f, p2f, so3, do3)
    return jnp.stack([o0[:E], o1[:E]], axis=1)


# speedup vs baseline: 22.5766x; 1.2475x over previous
import functools

import jax
import jax.numpy as jnp
from jax import lax
from jax.experimental import pallas as pl
from jax.experimental.pallas import tpu as pltpu
from jax.experimental.pallas import tpu_sc as plsc

N = 10000
E = 320000
D = 128
NLAYERS = 6
NPAD = 10240
NW = 32
K = 128
CH = 80
EPAD = NW * CH * K
KO = 128
CHO = EPAD // (NW * KO)
RPT = NPAD // 16

_mesh = plsc.VectorSubcoreMesh(core_axis_name="c", subcore_axis_name="s")


@functools.partial(
    pl.kernel,
    out_type=(jax.ShapeDtypeStruct((2, 1, NPAD), jnp.float32),
              jax.ShapeDtypeStruct((EPAD // 2,), jnp.int32),
              jax.ShapeDtypeStruct((EPAD // 2,), jnp.int32)),
    mesh=_mesh,
    compiler_params=pltpu.CompilerParams(needs_layout_passes=False),
    scratch_types=[
        pltpu.VMEM((CHO, KO), jnp.int32),
        pltpu.VMEM((CHO, KO), jnp.int32),
        pltpu.VMEM((CHO * KO // 2,), jnp.int32),
        pltpu.VMEM((CHO * KO // 2,), jnp.int32),
        pltpu.VMEM((KO,), jnp.float32),
        pltpu.VMEM((RPT,), jnp.float32),
        pltpu.VMEM_SHARED((NPAD,), jnp.float32),
    ],
)
def _sc_degree(dst_hbm, src_hbm, out_hbm, pks_hbm, pkd_hbm,
               dst_v, src_v, pkd_v, pks_v, ones_v, buf_v, acc):
    c = lax.axis_index("c")
    s = lax.axis_index("s")
    wid = c * 16 + s
    pltpu.sync_copy(dst_hbm.at[wid], dst_v)
    pltpu.sync_copy(src_hbm.at[wid], src_v)

    def pbody(j, carry):
        for v, pk in ((src_v, pks_v), (dst_v, pkd_v)):
            for g in range(KO // 32):
                w0 = v[j, pl.ds(g * 32, 16)]
                w1 = v[j, pl.ds(g * 32 + 16, 16)]
                pk[pl.ds(j * (KO // 2) + g * 16, 16)] = w0 | (w1 << 16)
        return carry

    lax.fori_loop(0, CHO, pbody, 0)
    poff = pl.multiple_of(wid * (CHO * KO // 2), 128)
    pltpu.sync_copy(pks_v, pks_hbm.at[pl.ds(poff, CHO * KO // 2)])
    pltpu.sync_copy(pkd_v, pkd_hbm.at[pl.ds(poff, CHO * KO // 2)])

    for i in range(KO // 16):
        ones_v[pl.ds(i * 16, 16)] = jnp.ones((16,), jnp.float32)
    for i in range(RPT // 16):
        buf_v[pl.ds(i * 16, 16)] = jnp.zeros((16,), jnp.float32)
    roff = pl.multiple_of(s * RPT, 128)
    pltpu.sync_copy(buf_v, acc.at[pl.ds(roff, RPT)])
    plsc.subcore_barrier()

    def body(j, carry):
        pltpu.sync_copy(ones_v, acc.at[dst_v.at[j]], add=True)
        return carry

    lax.fori_loop(0, CHO, body, 0)
    plsc.subcore_barrier()
    pltpu.sync_copy(acc.at[pl.ds(roff, RPT)], buf_v)
    pltpu.sync_copy(buf_v, out_hbm.at[c, 0, pl.ds(roff, RPT)])


@functools.partial(
    pl.kernel,
    out_type=(jax.ShapeDtypeStruct((NPAD, D), jnp.float32),
              jax.ShapeDtypeStruct((NPAD, D), jnp.float32)),
    mesh=_mesh,
    compiler_params=pltpu.CompilerParams(needs_layout_passes=False),
    scratch_types=[
        pltpu.VMEM((CH * K // 2,), jnp.int32),
        pltpu.VMEM((CH * K // 2,), jnp.int32),
        pltpu.VMEM((2, K), jnp.int32),
        pltpu.VMEM((2, K), jnp.int32),
        pltpu.VMEM((2, K, D), jnp.float32),
        pltpu.VMEM_SHARED((NPAD, D), jnp.float32),
        pltpu.SemaphoreType.DMA,
        pltpu.SemaphoreType.DMA,
    ],
)
def _sc_segsum(h_hbm, src_hbm, dst_hbm, out0_hbm, out1_hbm,
               src16_v, dst16_v, sidx, didx, rowbuf, acc, sem0, sem1):
    c = lax.axis_index("c")
    s = lax.axis_index("s")
    wid = c * 16 + s
    ioff = pl.multiple_of(wid * (CH * K // 2), 128)
    pltpu.sync_copy(src_hbm.at[pl.ds(ioff, CH * K // 2)], src16_v)
    pltpu.sync_copy(dst_hbm.at[pl.ds(ioff, CH * K // 2)], dst16_v)

    def cvt(j, b):
        for v32, ring in ((src16_v, sidx), (dst16_v, didx)):
            for g in range(K // 32):
                w = v32[pl.ds(j * (K // 2) + g * 16, 16)]
                ring[b, pl.ds(g * 32, 16)] = w & 0xFFFF
                ring[b, pl.ds(g * 32 + 16, 16)] = lax.shift_right_logical(
                    w, 16)

    def zb(t, carry):
        rowbuf[0, t // 8, pl.ds((t % 8) * 16, 16)] = jnp.zeros((16,),
                                                               jnp.float32)
        return carry

    lax.fori_loop(0, K * (D // 16), zb, 0)
    roff = pl.multiple_of(s * RPT, 128)
    for i in range(RPT // K):
        pltpu.sync_copy(rowbuf.at[0], acc.at[pl.ds(roff + i * K, K), :])
    plsc.subcore_barrier()

    sems = (sem0, sem1)
    cvt(0, 0)
    pltpu.async_copy(h_hbm.at[sidx.at[0]], rowbuf.at[0], sem0)
    cvt(1, 1)
    pltpu.async_copy(h_hbm.at[sidx.at[1]], rowbuf.at[1], sem1)

    def body(i, carry):
        j = i * 2
        for b in range(2):
            jj = j + b
            pltpu.make_async_copy(h_hbm.at[sidx.at[b]], rowbuf.at[b],
                                  sems[b]).wait()
            pltpu.sync_copy(rowbuf.at[b], acc.at[didx.at[b]], add=True)

            @pl.when(jj + 2 < CH)
            def _():
                cvt(jj + 2, b)
                pltpu.async_copy(h_hbm.at[sidx.at[b]], rowbuf.at[b],
                                 sems[b])

        return carry

    lax.fori_loop(0, CH // 2, body, 0)
    plsc.subcore_barrier()
    for i in range(RPT // K):
        pltpu.sync_copy(acc.at[pl.ds(roff + i * K, K), :], rowbuf.at[0])

        @pl.when(c == 0)
        def _():
            pltpu.sync_copy(rowbuf.at[0],
                            out0_hbm.at[pl.ds(roff + i * K, K), :])

        @pl.when(c == 1)
        def _():
            pltpu.sync_copy(rowbuf.at[0],
                            out1_hbm.at[pl.ds(roff + i * K, K), :])


@functools.partial(
    pl.kernel,
    out_type=(jax.ShapeDtypeStruct((EPAD,), jnp.float32),
              jax.ShapeDtypeStruct((EPAD,), jnp.float32)),
    mesh=_mesh,
    compiler_params=pltpu.CompilerParams(needs_layout_passes=False),
    scratch_types=[
        pltpu.VMEM((CHO, KO), jnp.int32),
        pltpu.VMEM((CHO, KO), jnp.int32),
        pltpu.VMEM((2 * NPAD,), jnp.float32),
        pltpu.VMEM((2 * NPAD,), jnp.float32),
        pltpu.VMEM((KO,), jnp.float32),
        pltpu.VMEM((KO,), jnp.float32),
    ],
)
def _sc_edgeout(p1_hbm, p2_hbm, s_hbm, d_hbm, out0_hbm, out1_hbm,
                s_v, d_v, p1_v, p2_v, obuf0, obuf1):
    c = lax.axis_index("c")
    s = lax.axis_index("s")
    wid = c * 16 + s
    pltpu.sync_copy(s_hbm.at[wid], s_v)
    pltpu.sync_copy(d_hbm.at[wid], d_v)
    pltpu.sync_copy(p1_hbm, p1_v)
    pltpu.sync_copy(p2_hbm, p2_v)

    def body(j, carry):
        for g in range(KO // 16):
            sseg = s_v[j, pl.ds(g * 16, 16)]
            dseg = d_v[j, pl.ds(g * 16, 16)]
            a0 = plsc.load_gather(p1_v, [sseg * 2])
            a1 = plsc.load_gather(p1_v, [sseg * 2 + 1])
            b0 = plsc.load_gather(p2_v, [dseg * 2])
            b1 = plsc.load_gather(p2_v, [dseg * 2 + 1])
            obuf0[pl.ds(g * 16, 16)] = a0 + b0
            obuf1[pl.ds(g * 16, 16)] = a1 + b1
        base = pl.multiple_of((wid * CHO + j) * KO, 128)
        pltpu.sync_copy(obuf0, out0_hbm.at[pl.ds(base, KO)])
        pltpu.sync_copy(obuf1, out1_hbm.at[pl.ds(base, KO)])
        return carry

    lax.fori_loop(0, CHO, body, 0)


def _tc_embed_body(x_ref, wemb_ref, bemb_ref, wc0_ref, degt_ref,
                   xe_ref, dinvb_ref, h_ref):
    i = pl.program_id(0)
    xe = jnp.dot(x_ref[...], wemb_ref[...],
                 preferred_element_type=jnp.float32) + bemb_ref[...]
    rows = i * 256 + lax.broadcasted_iota(jnp.int32, (256, 1), 0)
    valid = rows < N
    xe = jnp.where(valid, xe, 0.0)
    deg = degt_ref[...]
    dtot = deg[:, 0:1] + deg[:, 1:2] + 1.0
    dinv = jnp.where(valid, 1.0 / jnp.sqrt(dtot), 0.0)
    dinvb = jnp.broadcast_to(dinv, (256, D))
    xe_ref[...] = xe
    dinvb_ref[...] = dinvb
    h_ref[...] = jnp.dot(xe * dinvb, wc0_ref[...],
                         preferred_element_type=jnp.float32)


_tc_embed = pl.pallas_call(
    _tc_embed_body,
    grid=(NPAD // 256,),
    in_specs=[
        pl.BlockSpec((256, D), lambda i: (i, 0)),
        pl.BlockSpec((D, D), lambda i: (0, 0)),
        pl.BlockSpec((1, D), lambda i: (0, 0)),
        pl.BlockSpec((D, D), lambda i: (0, 0)),
        pl.BlockSpec((256, 2), lambda i: (i, 0)),
    ],
    out_specs=[
        pl.BlockSpec((256, D), lambda i: (i, 0)),
        pl.BlockSpec((256, D), lambda i: (i, 0)),
        pl.BlockSpec((256, D), lambda i: (i, 0)),
    ],
    out_shape=[
        jax.ShapeDtypeStruct((NPAD, D), jnp.float32),
        jax.ShapeDtypeStruct((NPAD, D), jnp.float32),
        jax.ShapeDtypeStruct((NPAD, D), jnp.float32),
    ],
)


def _combine_core(s0_ref, s1_ref, h_ref, xe_ref, dinvb_ref,
                  bc_ref, g_ref, b_ref):
    agg = (s0_ref[...] + s1_ref[...] + h_ref[...]) * dinvb_ref[...] \
        + bc_ref[...]
    rows = lax.broadcasted_iota(jnp.int32, (NPAD, 1), 0)
    valid = rows < N
    agg = jnp.where(valid, agg, 0.0)
    mu = jnp.sum(agg, axis=0, keepdims=True) / N
    var = jnp.sum(agg * agg, axis=0, keepdims=True) / N - mu * mu
    hbn = (agg - mu) * (1.0 / jnp.sqrt(var + 1e-5)) * g_ref[...] + b_ref[...]
    xen = xe_ref[...] + jnp.maximum(hbn, 0.0)
    return jnp.where(valid, xen, 0.0)


def _tc_combine_body(s0_ref, s1_ref, h_ref, xe_ref, dinvb_ref,
                     bc_ref, g_ref, b_ref, wn_ref, xe_out, hn_out):
    xen = _combine_core(s0_ref, s1_ref, h_ref, xe_ref, dinvb_ref,
                        bc_ref, g_ref, b_ref)
    xe_out[...] = xen
    hn_out[...] = jnp.dot(xen * dinvb_ref[...], wn_ref[...],
                          preferred_element_type=jnp.float32)


_tc_combine = pl.pallas_call(
    _tc_combine_body,
    out_shape=[
        jax.ShapeDtypeStruct((NPAD, D), jnp.float32),
        jax.ShapeDtypeStruct((NPAD, D), jnp.float32),
    ],
)


def _tc_combine_last_body(s0_ref, s1_ref, h_ref, xe_ref, dinvb_ref,
                          bc_ref, g_ref, b_ref, wfc_ref, p4_out):
    xen = _combine_core(s0_ref, s1_ref, h_ref, xe_ref, dinvb_ref,
                        bc_ref, g_ref, b_ref)
    p4_out[...] = jnp.dot(xen, wfc_ref[...],
                          preferred_element_type=jnp.float32)


_tc_combine_last = pl.pallas_call(
    _tc_combine_last_body,
    out_shape=jax.ShapeDtypeStruct((NPAD, 4), jnp.float32),
)


def _pad_idx(a, kk):
    fill = N + (jnp.arange(EPAD - E, dtype=jnp.int32) % (NPAD - N))
    a = jnp.concatenate([a, fill])
    return a.reshape(NW, EPAD // (NW * kk), kk)


def kernel(x, edge_index, edge_index_out, W_emb, b_emb, Wc, bc, gamma, beta,
           W_fc, b_fc):
    x_pad = jnp.pad(x, ((0, NPAD - N), (0, 0)))
    src3w = _pad_idx(edge_index[0], KO)
    dst3w = _pad_idx(edge_index[1], KO)
    so3 = _pad_idx(edge_index_out[0], KO)
    do3 = _pad_idx(edge_index_out[1], KO)

    deg2, src3, dst3 = _sc_degree(dst3w, src3w)
    degt = jnp.transpose(deg2[:, 0, :])

    xe, dinvb, h = _tc_embed(x_pad, W_emb, b_emb.reshape(1, D), Wc[0], degt)

    for i in range(NLAYERS):
        s0, s1 = _sc_segsum(h, src3, dst3)
        bci = bc[i].reshape(1, D)
        gi = gamma[i].reshape(1, D)
        bi = beta[i].reshape(1, D)
        if i < NLAYERS - 1:
            xe, h = _tc_combine(s0, s1, h, xe, dinvb, bci, gi, bi,
                                Wc[i + 1])
        else:
            wfc4 = jnp.concatenate([W_fc[:D], W_fc[D:]], axis=1)
            p4 = _tc_combine_last(s0, s1, h, xe, dinvb, bci, gi, bi,
                                  wfc4)

    p1f = (p4[:, 0:2] + b_fc).reshape(-1)
    p2f = p4[:, 2:4].reshape(-1)
    o0, o1 = _sc_edgeout(p1f, p2f, so3, do3)
    return jnp.stack([o0[:E], o1[:E]], axis=1)


# speedup vs baseline: 23.2328x; 1.0291x over previous
import functools

import jax
import jax.numpy as jnp
from jax import lax
from jax.experimental import pallas as pl
from jax.experimental.pallas import tpu as pltpu
from jax.experimental.pallas import tpu_sc as plsc

N = 10000
E = 320000
D = 128
NLAYERS = 6
NPAD = 10240
NW = 32
K = 128
CH = 80
EPAD = NW * CH * K
KO = 128
CHO = EPAD // (NW * KO)
RPT = NPAD // 16

_mesh = plsc.VectorSubcoreMesh(core_axis_name="c", subcore_axis_name="s")


@functools.partial(
    pl.kernel,
    out_type=(jax.ShapeDtypeStruct((2, 1, NPAD), jnp.float32),
              jax.ShapeDtypeStruct((EPAD // 2,), jnp.int32),
              jax.ShapeDtypeStruct((EPAD // 2,), jnp.int32)),
    mesh=_mesh,
    compiler_params=pltpu.CompilerParams(needs_layout_passes=False),
    scratch_types=[
        pltpu.VMEM((CHO, KO), jnp.int32),
        pltpu.VMEM((CHO, KO), jnp.int32),
        pltpu.VMEM((CHO * KO // 2,), jnp.int32),
        pltpu.VMEM((CHO * KO // 2,), jnp.int32),
        pltpu.VMEM((KO,), jnp.float32),
        pltpu.VMEM((RPT,), jnp.float32),
        pltpu.VMEM_SHARED((NPAD,), jnp.float32),
    ],
)
def _sc_degree(dst_hbm, src_hbm, out_hbm, pks_hbm, pkd_hbm,
               dst_v, src_v, pkd_v, pks_v, ones_v, buf_v, acc):
    c = lax.axis_index("c")
    s = lax.axis_index("s")
    wid = c * 16 + s
    pltpu.sync_copy(dst_hbm.at[wid], dst_v)
    pltpu.sync_copy(src_hbm.at[wid], src_v)

    def pbody(j, carry):
        for v, pk in ((src_v, pks_v), (dst_v, pkd_v)):
            for g in range(KO // 32):
                w0 = v[j, pl.ds(g * 32, 16)]
                w1 = v[j, pl.ds(g * 32 + 16, 16)]
                pk[pl.ds(j * (KO // 2) + g * 16, 16)] = w0 | (w1 << 16)
        return carry

    lax.fori_loop(0, CHO, pbody, 0)
    poff = pl.multiple_of(wid * (CHO * KO // 2), 128)
    pltpu.sync_copy(pks_v, pks_hbm.at[pl.ds(poff, CHO * KO // 2)])
    pltpu.sync_copy(pkd_v, pkd_hbm.at[pl.ds(poff, CHO * KO // 2)])

    for i in range(KO // 16):
        ones_v[pl.ds(i * 16, 16)] = jnp.ones((16,), jnp.float32)
    for i in range(RPT // 16):
        buf_v[pl.ds(i * 16, 16)] = jnp.zeros((16,), jnp.float32)
    roff = pl.multiple_of(s * RPT, 128)
    pltpu.sync_copy(buf_v, acc.at[pl.ds(roff, RPT)])
    plsc.subcore_barrier()

    def body(j, carry):
        pltpu.sync_copy(ones_v, acc.at[dst_v.at[j]], add=True)
        return carry

    lax.fori_loop(0, CHO, body, 0)
    plsc.subcore_barrier()
    pltpu.sync_copy(acc.at[pl.ds(roff, RPT)], buf_v)
    pltpu.sync_copy(buf_v, out_hbm.at[c, 0, pl.ds(roff, RPT)])


KSEG = 64
CHS = EPAD // (NW * KSEG)


@functools.partial(
    pl.kernel,
    out_type=jax.ShapeDtypeStruct((2, NPAD, D), jnp.float32),
    mesh=_mesh,
    compiler_params=pltpu.CompilerParams(needs_layout_passes=False),
    scratch_types=[
        pltpu.VMEM((CHS * KSEG // 2,), jnp.int32),
        pltpu.VMEM((CHS * KSEG // 2,), jnp.int32),
        pltpu.VMEM((4, KSEG), jnp.int32),
        pltpu.VMEM((4, KSEG), jnp.int32),
        pltpu.VMEM((4, KSEG, D), jnp.float32),
        pltpu.VMEM_SHARED((NPAD, D), jnp.float32),
        pltpu.SemaphoreType.DMA,
        pltpu.SemaphoreType.DMA,
        pltpu.SemaphoreType.DMA,
        pltpu.SemaphoreType.DMA,
        pltpu.SemaphoreType.DMA,
        pltpu.SemaphoreType.DMA,
        pltpu.SemaphoreType.DMA,
        pltpu.SemaphoreType.DMA,
    ],
)
def _sc_segsum(h_hbm, src_hbm, dst_hbm, zeros_hbm, out_hbm,
               src16_v, dst16_v, sidx, didx, rowbuf, acc,
               g0, g1, g2, g3, s0, s1, s2, s3):
    c = lax.axis_index("c")
    s = lax.axis_index("s")
    wid = c * 16 + s
    gsem = (g0, g1, g2, g3)
    ssem = (s0, s1, s2, s3)
    NWRD = CHS * KSEG // 2
    ioff = pl.multiple_of(wid * NWRD, 128)
    pltpu.sync_copy(src_hbm.at[pl.ds(ioff, NWRD)], src16_v)
    pltpu.sync_copy(dst_hbm.at[pl.ds(ioff, NWRD)], dst16_v)

    def cvt(j, b):
        for v32, ring in ((src16_v, sidx), (dst16_v, didx)):
            for g in range(KSEG // 32):
                w = v32[pl.ds(j * (KSEG // 2) + g * 16, 16)]
                ring[b, pl.ds(g * 32, 16)] = w & 0xFFFF
                ring[b, pl.ds(g * 32 + 16, 16)] = lax.shift_right_logical(
                    w, 16)

    pltpu.sync_copy(zeros_hbm, rowbuf.at[0])
    roff = pl.multiple_of(s * RPT, 128)
    NZ = RPT // KSEG
    for i in range(NZ):
        pltpu.async_copy(rowbuf.at[0],
                         acc.at[pl.ds(roff + i * KSEG, KSEG), :],
                         ssem[i % 4])
    for i in range(NZ):
        pltpu.make_async_copy(rowbuf.at[0],
                              acc.at[pl.ds(roff + i * KSEG, KSEG), :],
                              ssem[i % 4]).wait()
    plsc.subcore_barrier()

    def wait_g(b):
        pltpu.make_async_copy(h_hbm.at[sidx.at[b]], rowbuf.at[b],
                              gsem[b]).wait()

    def issue_s(b):
        pltpu.async_copy(rowbuf.at[b], acc.at[didx.at[b]], ssem[b], add=True)

    def wait_s(b):
        pltpu.make_async_copy(rowbuf.at[b], acc.at[didx.at[b]],
                              ssem[b]).wait()

    def prep(j, b):
        cvt(j, b)
        pltpu.async_copy(h_hbm.at[sidx.at[b]], rowbuf.at[b], gsem[b])

    for b in range(3):
        prep(b, b)
    for jj in range(4):
        wait_g(jj)
        issue_s(jj)
        if jj == 0:
            prep(3, 3)
        else:
            bf = (jj + 3) % 4
            wait_s(bf)
            prep(jj + 3, bf)

    def body(i, carry):
        j = i * 4
        for b in range(4):
            jj = j + b
            bf = (b + 3) % 4
            wait_g(b)
            issue_s(b)
            wait_s(bf)
            prep(jj + 3, bf)
        return carry

    lax.fori_loop(1, CHS // 4 - 1, body, 0)
    for b in range(4):
        jj = CHS - 4 + b
        wait_g(b)
        issue_s(b)
        if b == 0:
            wait_s(3)
            prep(CHS - 1, 3)
    for b in range(4):
        wait_s(b)
    plsc.subcore_barrier()
    NZ2 = RPT // K
    for i in range(NZ2):
        pltpu.async_copy(acc.at[pl.ds(roff + i * K, K), :],
                         out_hbm.at[c, pl.ds(roff + i * K, K), :],
                         gsem[i % 4])
    for i in range(NZ2):
        pltpu.make_async_copy(acc.at[pl.ds(roff + i * K, K), :],
                              out_hbm.at[c, pl.ds(roff + i * K, K), :],
                              gsem[i % 4]).wait()


@functools.partial(
    pl.kernel,
    out_type=(jax.ShapeDtypeStruct((EPAD,), jnp.float32),
              jax.ShapeDtypeStruct((EPAD,), jnp.float32)),
    mesh=_mesh,
    compiler_params=pltpu.CompilerParams(needs_layout_passes=False),
    scratch_types=[
        pltpu.VMEM((CHO, KO), jnp.int32),
        pltpu.VMEM((CHO, KO), jnp.int32),
        pltpu.VMEM((2 * NPAD,), jnp.float32),
        pltpu.VMEM((2 * NPAD,), jnp.float32),
        pltpu.VMEM((KO,), jnp.float32),
        pltpu.VMEM((KO,), jnp.float32),
    ],
)
def _sc_edgeout(p1_hbm, p2_hbm, s_hbm, d_hbm, out0_hbm, out1_hbm,
                s_v, d_v, p1_v, p2_v, obuf0, obuf1):
    c = lax.axis_index("c")
    s = lax.axis_index("s")
    wid = c * 16 + s
    pltpu.sync_copy(s_hbm.at[wid], s_v)
    pltpu.sync_copy(d_hbm.at[wid], d_v)
    pltpu.sync_copy(p1_hbm, p1_v)
    pltpu.sync_copy(p2_hbm, p2_v)

    def body(j, carry):
        for g in range(KO // 16):
            sseg = s_v[j, pl.ds(g * 16, 16)]
            dseg = d_v[j, pl.ds(g * 16, 16)]
            a0 = plsc.load_gather(p1_v, [sseg * 2])
            a1 = plsc.load_gather(p1_v, [sseg * 2 + 1])
            b0 = plsc.load_gather(p2_v, [dseg * 2])
            b1 = plsc.load_gather(p2_v, [dseg * 2 + 1])
            obuf0[pl.ds(g * 16, 16)] = a0 + b0
            obuf1[pl.ds(g * 16, 16)] = a1 + b1
        base = pl.multiple_of((wid * CHO + j) * KO, 128)
        pltpu.sync_copy(obuf0, out0_hbm.at[pl.ds(base, KO)])
        pltpu.sync_copy(obuf1, out1_hbm.at[pl.ds(base, KO)])
        return carry

    lax.fori_loop(0, CHO, body, 0)


def _tc_embed_body(x_ref, wemb_ref, bemb_ref, wc0_ref, degt_ref,
                   xe_ref, dinvb_ref, h_ref):
    i = pl.program_id(0)
    xe = jnp.dot(x_ref[...], wemb_ref[...],
                 preferred_element_type=jnp.float32) + bemb_ref[...]
    rows = i * 256 + lax.broadcasted_iota(jnp.int32, (256, 1), 0)
    valid = rows < N
    xe = jnp.where(valid, xe, 0.0)
    deg = degt_ref[...]
    dtot = deg[:, 0:1] + deg[:, 1:2] + 1.0
    dinv = jnp.where(valid, 1.0 / jnp.sqrt(dtot), 0.0)
    dinvb = jnp.broadcast_to(dinv, (256, D))
    xe_ref[...] = xe
    dinvb_ref[...] = dinvb
    h_ref[...] = jnp.dot(xe * dinvb, wc0_ref[...],
                         preferred_element_type=jnp.float32)


_tc_embed = pl.pallas_call(
    _tc_embed_body,
    grid=(NPAD // 256,),
    in_specs=[
        pl.BlockSpec((256, D), lambda i: (i, 0)),
        pl.BlockSpec((D, D), lambda i: (0, 0)),
        pl.BlockSpec((1, D), lambda i: (0, 0)),
        pl.BlockSpec((D, D), lambda i: (0, 0)),
        pl.BlockSpec((256, 2), lambda i: (i, 0)),
    ],
    out_specs=[
        pl.BlockSpec((256, D), lambda i: (i, 0)),
        pl.BlockSpec((256, D), lambda i: (i, 0)),
        pl.BlockSpec((256, D), lambda i: (i, 0)),
    ],
    out_shape=[
        jax.ShapeDtypeStruct((NPAD, D), jnp.float32),
        jax.ShapeDtypeStruct((NPAD, D), jnp.float32),
        jax.ShapeDtypeStruct((NPAD, D), jnp.float32),
    ],
)


def _combine_core(s2_ref, h_ref, xe_ref, dinvb_ref,
                  bc_ref, g_ref, b_ref):
    agg = (s2_ref[0] + s2_ref[1] + h_ref[...]) * dinvb_ref[...] \
        + bc_ref[...]
    rows = lax.broadcasted_iota(jnp.int32, (NPAD, 1), 0)
    valid = rows < N
    agg = jnp.where(valid, agg, 0.0)
    mu = jnp.sum(agg, axis=0, keepdims=True) / N
    var = jnp.sum(agg * agg, axis=0, keepdims=True) / N - mu * mu
    hbn = (agg - mu) * (1.0 / jnp.sqrt(var + 1e-5)) * g_ref[...] + b_ref[...]
    xen = xe_ref[...] + jnp.maximum(hbn, 0.0)
    return jnp.where(valid, xen, 0.0)


def _tc_combine_body(s2_ref, h_ref, xe_ref, dinvb_ref,
                     bc_ref, g_ref, b_ref, wn_ref, xe_out, hn_out):
    xen = _combine_core(s2_ref, h_ref, xe_ref, dinvb_ref,
                        bc_ref, g_ref, b_ref)
    xe_out[...] = xen
    hn_out[...] = jnp.dot(xen * dinvb_ref[...], wn_ref[...],
                          preferred_element_type=jnp.float32)


_tc_combine = pl.pallas_call(
    _tc_combine_body,
    out_shape=[
        jax.ShapeDtypeStruct((NPAD, D), jnp.float32),
        jax.ShapeDtypeStruct((NPAD, D), jnp.float32),
    ],
)


def _tc_combine_last_body(s2_ref, h_ref, xe_ref, dinvb_ref,
                          bc_ref, g_ref, b_ref, wfc_ref, p4_out):
    xen = _combine_core(s2_ref, h_ref, xe_ref, dinvb_ref,
                        bc_ref, g_ref, b_ref)
    p4_out[...] = jnp.dot(xen, wfc_ref[...],
                          preferred_element_type=jnp.float32)


_tc_combine_last = pl.pallas_call(
    _tc_combine_last_body,
    out_shape=jax.ShapeDtypeStruct((NPAD, 4), jnp.float32),
)


def _pad_idx(a, kk):
    fill = N + (jnp.arange(EPAD - E, dtype=jnp.int32) % (NPAD - N))
    a = jnp.concatenate([a, fill])
    return a.reshape(NW, EPAD // (NW * kk), kk)


def kernel(x, edge_index, edge_index_out, W_emb, b_emb, Wc, bc, gamma, beta,
           W_fc, b_fc):
    x_pad = jnp.pad(x, ((0, NPAD - N), (0, 0)))
    src3w = _pad_idx(edge_index[0], KO)
    dst3w = _pad_idx(edge_index[1], KO)
    so3 = _pad_idx(edge_index_out[0], KO)
    do3 = _pad_idx(edge_index_out[1], KO)

    deg2, src3, dst3 = _sc_degree(dst3w, src3w)
    degt = jnp.transpose(deg2[:, 0, :])

    xe, dinvb, h = _tc_embed(x_pad, W_emb, b_emb.reshape(1, D), Wc[0], degt)
    zkd = jnp.zeros((KSEG, D), jnp.float32)

    for i in range(NLAYERS):
        s2 = _sc_segsum(h, src3, dst3, zkd)
        bci = bc[i].reshape(1, D)
        gi = gamma[i].reshape(1, D)
        bi = beta[i].reshape(1, D)
        if i < NLAYERS - 1:
            xe, h = _tc_combine(s2, h, xe, dinvb, bci, gi, bi,
                                Wc[i + 1])
        else:
            wfc4 = jnp.concatenate([W_fc[:D], W_fc[D:]], axis=1)
            p4 = _tc_combine_last(s2, h, xe, dinvb, bci, gi, bi,
                                  wfc4)

    p1f = (p4[:, 0:2] + b_fc).reshape(-1)
    p2f = p4[:, 2:4].reshape(-1)
    o0, o1 = _sc_edgeout(p1f, p2f, so3, do3)
    return jnp.stack([o0[:E], o1[:E]], axis=1)


# speedup vs baseline: 23.4817x; 1.0107x over previous
import functools

import jax
import jax.numpy as jnp
from jax import lax
from jax.experimental import pallas as pl
from jax.experimental.pallas import tpu as pltpu
from jax.experimental.pallas import tpu_sc as plsc

N = 10000
E = 320000
D = 128
NLAYERS = 6
NPAD = 10240
NW = 32
K = 128
CH = 80
EPAD = NW * CH * K
KO = 128
CHO = EPAD // (NW * KO)
RPT = NPAD // 16

_mesh = plsc.VectorSubcoreMesh(core_axis_name="c", subcore_axis_name="s")


@functools.partial(
    pl.kernel,
    out_type=(jax.ShapeDtypeStruct((2, 1, NPAD), jnp.float32),
              jax.ShapeDtypeStruct((EPAD // 2,), jnp.int32),
              jax.ShapeDtypeStruct((EPAD // 2,), jnp.int32)),
    mesh=_mesh,
    compiler_params=pltpu.CompilerParams(needs_layout_passes=False),
    scratch_types=[
        pltpu.VMEM((CHO, KO), jnp.int32),
        pltpu.VMEM((CHO, KO), jnp.int32),
        pltpu.VMEM((CHO * KO // 2,), jnp.int32),
        pltpu.VMEM((CHO * KO // 2,), jnp.int32),
        pltpu.VMEM((KO,), jnp.float32),
        pltpu.VMEM((RPT,), jnp.float32),
        pltpu.VMEM_SHARED((NPAD,), jnp.float32),
    ],
)
def _sc_degree(dst_hbm, src_hbm, out_hbm, pks_hbm, pkd_hbm,
               dst_v, src_v, pkd_v, pks_v, ones_v, buf_v, acc):
    c = lax.axis_index("c")
    s = lax.axis_index("s")
    wid = c * 16 + s
    pltpu.sync_copy(dst_hbm.at[wid], dst_v)
    pltpu.sync_copy(src_hbm.at[wid], src_v)

    def pbody(j, carry):
        for v, pk in ((src_v, pks_v), (dst_v, pkd_v)):
            for g in range(KO // 32):
                w0 = v[j, pl.ds(g * 32, 16)]
                w1 = v[j, pl.ds(g * 32 + 16, 16)]
                pk[pl.ds(j * (KO // 2) + g * 16, 16)] = w0 | (w1 << 16)
        return carry

    lax.fori_loop(0, CHO, pbody, 0)
    poff = pl.multiple_of(wid * (CHO * KO // 2), 128)
    pltpu.sync_copy(pks_v, pks_hbm.at[pl.ds(poff, CHO * KO // 2)])
    pltpu.sync_copy(pkd_v, pkd_hbm.at[pl.ds(poff, CHO * KO // 2)])

    for i in range(KO // 16):
        ones_v[pl.ds(i * 16, 16)] = jnp.ones((16,), jnp.float32)
    for i in range(RPT // 16):
        buf_v[pl.ds(i * 16, 16)] = jnp.zeros((16,), jnp.float32)
    roff = pl.multiple_of(s * RPT, 128)
    pltpu.sync_copy(buf_v, acc.at[pl.ds(roff, RPT)])
    plsc.subcore_barrier()

    def body(j, carry):
        pltpu.sync_copy(ones_v, acc.at[dst_v.at[j]], add=True)
        return carry

    lax.fori_loop(0, CHO, body, 0)
    plsc.subcore_barrier()
    pltpu.sync_copy(acc.at[pl.ds(roff, RPT)], buf_v)
    pltpu.sync_copy(buf_v, out_hbm.at[c, 0, pl.ds(roff, RPT)])


KSEG = 64
CHS = EPAD // (NW * KSEG)


@functools.partial(
    pl.kernel,
    out_type=jax.ShapeDtypeStruct((2, NPAD, D), jnp.float32),
    mesh=_mesh,
    compiler_params=pltpu.CompilerParams(needs_layout_passes=False),
    scratch_types=[
        pltpu.VMEM((CHS * KSEG // 2,), jnp.int32),
        pltpu.VMEM((CHS * KSEG // 2,), jnp.int32),
        pltpu.VMEM((4, KSEG), jnp.int32),
        pltpu.VMEM((4, KSEG), jnp.int32),
        pltpu.VMEM((4, KSEG, D), jnp.float32),
        pltpu.VMEM_SHARED((NPAD, D), jnp.float32),
        pltpu.SemaphoreType.DMA,
        pltpu.SemaphoreType.DMA,
        pltpu.SemaphoreType.DMA,
        pltpu.SemaphoreType.DMA,
        pltpu.SemaphoreType.DMA,
        pltpu.SemaphoreType.DMA,
        pltpu.SemaphoreType.DMA,
        pltpu.SemaphoreType.DMA,
    ],
)
def _sc_segsum(h_hbm, src_hbm, dst_hbm, zeros_hbm, out_hbm,
               src16_v, dst16_v, sidx, didx, rowbuf, acc,
               g0, g1, g2, g3, s0, s1, s2, s3):
    c = lax.axis_index("c")
    s = lax.axis_index("s")
    wid = c * 16 + s
    gsem = (g0, g1, g2, g3)
    ssem = (s0, s1, s2, s3)
    NWRD = CHS * KSEG // 2
    ioff = pl.multiple_of(wid * NWRD, 128)
    pltpu.sync_copy(src_hbm.at[pl.ds(ioff, NWRD)], src16_v)
    pltpu.sync_copy(dst_hbm.at[pl.ds(ioff, NWRD)], dst16_v)

    def cvt(j, b):
        for v32, ring in ((src16_v, sidx), (dst16_v, didx)):
            for g in range(KSEG // 32):
                w = v32[pl.ds(j * (KSEG // 2) + g * 16, 16)]
                ring[b, pl.ds(g * 32, 16)] = w & 0xFFFF
                ring[b, pl.ds(g * 32 + 16, 16)] = lax.shift_right_logical(
                    w, 16)

    pltpu.sync_copy(zeros_hbm, rowbuf.at[0])
    roff = pl.multiple_of(s * RPT, 128)
    NZ = RPT // KSEG
    for i in range(NZ):
        pltpu.async_copy(rowbuf.at[0],
                         acc.at[pl.ds(roff + i * KSEG, KSEG), :],
                         ssem[i % 4])
    for i in range(NZ):
        pltpu.make_async_copy(rowbuf.at[0],
                              acc.at[pl.ds(roff + i * KSEG, KSEG), :],
                              ssem[i % 4]).wait()
    plsc.subcore_barrier()

    def wait_g(b):
        pltpu.make_async_copy(h_hbm.at[sidx.at[b]], rowbuf.at[b],
                              gsem[b]).wait()

    def issue_s(b):
        pltpu.async_copy(rowbuf.at[b], acc.at[didx.at[b]], ssem[b], add=True)

    def wait_s(b):
        pltpu.make_async_copy(rowbuf.at[b], acc.at[didx.at[b]],
                              ssem[b]).wait()

    def prep(j, b):
        cvt(j, b)
        pltpu.async_copy(h_hbm.at[sidx.at[b]], rowbuf.at[b], gsem[b])

    for b in range(3):
        prep(b, b)
    for jj in range(4):
        wait_g(jj)
        issue_s(jj)
        if jj == 0:
            prep(3, 3)
        else:
            bf = (jj + 3) % 4
            wait_s(bf)
            prep(jj + 3, bf)

    def body(i, carry):
        j = i * 4
        for b in range(4):
            jj = j + b
            bf = (b + 3) % 4
            wait_g(b)
            issue_s(b)
            wait_s(bf)
            prep(jj + 3, bf)
        return carry

    lax.fori_loop(1, CHS // 4 - 1, body, 0)
    for b in range(4):
        jj = CHS - 4 + b
        wait_g(b)
        issue_s(b)
        if b == 0:
            wait_s(3)
            prep(CHS - 1, 3)
    for b in range(4):
        wait_s(b)
    plsc.subcore_barrier()
    NZ2 = RPT // K
    for i in range(NZ2):
        pltpu.async_copy(acc.at[pl.ds(roff + i * K, K), :],
                         out_hbm.at[c, pl.ds(roff + i * K, K), :],
                         gsem[i % 4])
    for i in range(NZ2):
        pltpu.make_async_copy(acc.at[pl.ds(roff + i * K, K), :],
                              out_hbm.at[c, pl.ds(roff + i * K, K), :],
                              gsem[i % 4]).wait()


@functools.partial(
    pl.kernel,
    out_type=(jax.ShapeDtypeStruct((EPAD,), jnp.float32),
              jax.ShapeDtypeStruct((EPAD,), jnp.float32)),
    mesh=_mesh,
    compiler_params=pltpu.CompilerParams(needs_layout_passes=False),
    scratch_types=[
        pltpu.VMEM((CHO, KO), jnp.int32),
        pltpu.VMEM((CHO, KO), jnp.int32),
        pltpu.VMEM((4, NPAD), jnp.float32),
        pltpu.VMEM((KO,), jnp.float32),
        pltpu.VMEM((KO,), jnp.float32),
    ],
)
def _sc_edgeout(pt_hbm, s_hbm, d_hbm, out0_hbm, out1_hbm,
                s_v, d_v, pt_v, obuf0, obuf1):
    c = lax.axis_index("c")
    s = lax.axis_index("s")
    wid = c * 16 + s
    pltpu.sync_copy(s_hbm.at[wid], s_v)
    pltpu.sync_copy(d_hbm.at[wid], d_v)
    pltpu.sync_copy(pt_hbm, pt_v)
    z16 = jnp.zeros((16,), jnp.int32)

    def body(j, carry):
        for g in range(KO // 16):
            sseg = s_v[j, pl.ds(g * 16, 16)]
            dseg = d_v[j, pl.ds(g * 16, 16)]
            a0 = plsc.load_gather(pt_v, [z16, sseg])
            a1 = plsc.load_gather(pt_v, [z16 + 1, sseg])
            b0 = plsc.load_gather(pt_v, [z16 + 2, dseg])
            b1 = plsc.load_gather(pt_v, [z16 + 3, dseg])
            obuf0[pl.ds(g * 16, 16)] = a0 + b0
            obuf1[pl.ds(g * 16, 16)] = a1 + b1
        base = pl.multiple_of((wid * CHO + j) * KO, 128)
        pltpu.sync_copy(obuf0, out0_hbm.at[pl.ds(base, KO)])
        pltpu.sync_copy(obuf1, out1_hbm.at[pl.ds(base, KO)])
        return carry

    lax.fori_loop(0, CHO, body, 0)


def _tc_embed_body(x_ref, wemb_ref, bemb_ref, wc0_ref, degt_ref,
                   xe_ref, h_ref):
    i = pl.program_id(0)
    xe = jnp.dot(x_ref[...], wemb_ref[...],
                 preferred_element_type=jnp.float32) + bemb_ref[...]
    rows = i * 256 + lax.broadcasted_iota(jnp.int32, (256, 1), 0)
    valid = rows < N
    xe = jnp.where(valid, xe, 0.0)
    deg = degt_ref[...]
    dtot = deg[:, 0:1] + deg[:, 1:2] + 1.0
    dinv = jnp.where(valid, 1.0 / jnp.sqrt(dtot), 0.0)
    xe_ref[...] = xe
    h_ref[...] = jnp.dot(xe * dinv, wc0_ref[...],
                         preferred_element_type=jnp.float32)


_tc_embed = pl.pallas_call(
    _tc_embed_body,
    grid=(NPAD // 256,),
    in_specs=[
        pl.BlockSpec((256, D), lambda i: (i, 0)),
        pl.BlockSpec((D, D), lambda i: (0, 0)),
        pl.BlockSpec((1, D), lambda i: (0, 0)),
        pl.BlockSpec((D, D), lambda i: (0, 0)),
        pl.BlockSpec((256, 2), lambda i: (i, 0)),
    ],
    out_specs=[
        pl.BlockSpec((256, D), lambda i: (i, 0)),
        pl.BlockSpec((256, D), lambda i: (i, 0)),
    ],
    out_shape=[
        jax.ShapeDtypeStruct((NPAD, D), jnp.float32),
        jax.ShapeDtypeStruct((NPAD, D), jnp.float32),
    ],
)


def _combine_core(s2_ref, h_ref, xe_ref, degt_ref,
                  bc_ref, g_ref, b_ref):
    rows = lax.broadcasted_iota(jnp.int32, (NPAD, 1), 0)
    valid = rows < N
    deg = degt_ref[...]
    dtot = deg[:, 0:1] + deg[:, 1:2] + 1.0
    dinv = jnp.where(valid, 1.0 / jnp.sqrt(dtot), 0.0)
    agg = (s2_ref[0] + s2_ref[1] + h_ref[...]) * dinv + bc_ref[...]
    agg = jnp.where(valid, agg, 0.0)
    mu = jnp.sum(agg, axis=0, keepdims=True) / N
    var = jnp.sum(agg * agg, axis=0, keepdims=True) / N - mu * mu
    hbn = (agg - mu) * (1.0 / jnp.sqrt(var + 1e-5)) * g_ref[...] + b_ref[...]
    xen = xe_ref[...] + jnp.maximum(hbn, 0.0)
    return jnp.where(valid, xen, 0.0), dinv


def _tc_combine_body(s2_ref, h_ref, xe_ref, degt_ref,
                     bc_ref, g_ref, b_ref, wn_ref, xe_out, hn_out):
    xen, dinv = _combine_core(s2_ref, h_ref, xe_ref, degt_ref,
                              bc_ref, g_ref, b_ref)
    xe_out[...] = xen
    hn_out[...] = jnp.dot(xen * dinv, wn_ref[...],
                          preferred_element_type=jnp.float32)


_tc_combine = pl.pallas_call(
    _tc_combine_body,
    out_shape=[
        jax.ShapeDtypeStruct((NPAD, D), jnp.float32),
        jax.ShapeDtypeStruct((NPAD, D), jnp.float32),
    ],
)


def _tc_combine_last_body(s2_ref, h_ref, xe_ref, degt_ref,
                          bc_ref, g_ref, b_ref, wfc_ref, bvec_ref, p4_out):
    xen, _ = _combine_core(s2_ref, h_ref, xe_ref, degt_ref,
                           bc_ref, g_ref, b_ref)
    p4_out[...] = lax.dot_general(
        wfc_ref[...], xen, (((0,), (1,)), ((), ())),
        preferred_element_type=jnp.float32) + bvec_ref[...]


_tc_combine_last = pl.pallas_call(
    _tc_combine_last_body,
    out_shape=jax.ShapeDtypeStruct((4, NPAD), jnp.float32),
)


def _pad_idx(a, kk):
    fill = N + (jnp.arange(EPAD - E, dtype=jnp.int32) % (NPAD - N))
    a = jnp.concatenate([a, fill])
    return a.reshape(NW, EPAD // (NW * kk), kk)


def kernel(x, edge_index, edge_index_out, W_emb, b_emb, Wc, bc, gamma, beta,
           W_fc, b_fc):
    x_pad = jnp.pad(x, ((0, NPAD - N), (0, 0)))
    src3w = _pad_idx(edge_index[0], KO)
    dst3w = _pad_idx(edge_index[1], KO)
    so3 = _pad_idx(edge_index_out[0], KO)
    do3 = _pad_idx(edge_index_out[1], KO)

    deg2, src3, dst3 = _sc_degree(dst3w, src3w)
    degt = jnp.transpose(deg2[:, 0, :])

    xe, h = _tc_embed(x_pad, W_emb, b_emb.reshape(1, D), Wc[0], degt)
    zkd = jnp.zeros((KSEG, D), jnp.float32)

    for i in range(NLAYERS):
        s2 = _sc_segsum(h, src3, dst3, zkd)
        bci = bc[i].reshape(1, D)
        gi = gamma[i].reshape(1, D)
        bi = beta[i].reshape(1, D)
        if i < NLAYERS - 1:
            xe, h = _tc_combine(s2, h, xe, degt, bci, gi, bi,
                                Wc[i + 1])
        else:
            wfc4 = jnp.concatenate([W_fc[:D], W_fc[D:]], axis=1)
            bvec = jnp.concatenate([b_fc, jnp.zeros((2,), jnp.float32)])
            p4t = _tc_combine_last(s2, h, xe, degt, bci, gi, bi,
                                   wfc4, bvec.reshape(4, 1))

    o0, o1 = _sc_edgeout(p4t, so3, do3)
    return jnp.stack([o0[:E], o1[:E]], axis=1)
